# Initial kernel scaffold; baseline (speedup 1.0000x reference)
import jax
import jax.numpy as jnp
from jax.experimental import pallas as pl


def kernel(x, edge_index, W1l, b1, W1r, W2l, b2, W2r):
    raise NotImplementedError("write your pallas kernel here")


# speedup vs baseline: 2.2398x; 2.2398x over previous
import functools

import jax
import jax.numpy as jnp
from jax import lax
from jax.experimental import pallas as pl
from jax.experimental.pallas import tpu as pltpu
from jax.experimental.pallas import tpu_sc as plsc

N = 10000
E = 160000
DIN = 256
H = 512

NC = 2
NS = 16
CH = 128
PER_TILE = 10240
E_PAD = PER_TILE * NS
CHUNKS = PER_TILE // CH
ROWS_PER_TILE = 632
N_ACC = ROWS_PER_TILE * NS
CW = 16


def _make_agg(with_counts):
    mesh = plsc.VectorSubcoreMesh(
        core_axis_name="c", subcore_axis_name="s", num_cores=NC, num_subcores=NS
    )
    out_type = jax.ShapeDtypeStruct((4, N_ACC, 128), jnp.float32)
    scratch = [
        pltpu.VMEM_SHARED((N_ACC, 128), jnp.float32),
        pltpu.VMEM((CH,), jnp.int32),
        pltpu.VMEM((CH,), jnp.int32),
        pltpu.VMEM((CH, 128), jnp.float32),
        pltpu.SemaphoreType.DMA,
    ]

    def body(table, src_all, dst, zrows, ones, agg_out,
             acc, src_v, dst_v, rows_v, sem):
        c = lax.axis_index("c")
        s = lax.axis_index("s")
        tile_base = s * PER_TILE
        row0 = s * ROWS_PER_TILE

        for p in range(2):
            count_pass = with_counts and p == 1
            if with_counts:
                j_table = c
                j_out = c + 2 * p
            else:
                j_table = c * 2 + p
                j_out = j_table
            pltpu.sync_copy(zrows, acc.at[pl.ds(row0, ROWS_PER_TILE)])
            if count_pass:
                pltpu.sync_copy(ones, rows_v)
            plsc.subcore_barrier()

            def chunk(k, carry):
                off = tile_base + k * CH
                pltpu.sync_copy(dst.at[pl.ds(off, CH)], dst_v)
                if not count_pass:
                    pltpu.sync_copy(
                        src_all.at[pl.ds(j_table * E_PAD + off, CH)], src_v
                    )
                    pltpu.async_copy(table.at[src_v], rows_v, sem).wait()
                pltpu.sync_copy(rows_v, acc.at[dst_v], add=True)
                return carry

            lax.fori_loop(0, CHUNKS, chunk, 0)
            plsc.subcore_barrier()

            pltpu.sync_copy(
                acc.at[pl.ds(row0, ROWS_PER_TILE)],
                agg_out.at[j_out, pl.ds(row0, ROWS_PER_TILE)],
            )
            if p == 0:
                plsc.subcore_barrier()

    return pl.kernel(
        body,
        out_type=out_type,
        mesh=mesh,
        scratch_types=tuple(scratch),
    )


_make_agg = functools.lru_cache(maxsize=None)(_make_agg)


NB = 400
GRID = N // NB


def _mm1_body(agg_ref, cnt_ref, x_ref, w1l_ref, b1_ref, w1r_ref, ht_ref):
    inv = 1.0 / jnp.maximum(cnt_ref[0, :, 0:1], 1.0)
    h = jnp.dot(x_ref[...], w1r_ref[...], preferred_element_type=jnp.float32)
    h += b1_ref[...]
    for jc in range(DIN // 128):
        a = agg_ref[jc] * inv
        h += jnp.dot(a, w1l_ref[pl.ds(jc * 128, 128), :],
                     preferred_element_type=jnp.float32)
    h = jnp.maximum(h, 0.0)
    for jc in range(H // 128):
        ht_ref[jc] = h[:, jc * 128:(jc + 1) * 128]


def _mm2_body(agg_ref, cnt_ref, ht_ref, w2l_ref, b2_ref, w2r_ref, out_ref):
    inv = 1.0 / jnp.maximum(cnt_ref[0, :, 0:1], 1.0)
    o = jnp.broadcast_to(b2_ref[...], (NB, H)).astype(jnp.float32)
    for jc in range(H // 128):
        a = agg_ref[jc] * inv
        o += jnp.dot(a, w2l_ref[pl.ds(jc * 128, 128), :],
                     preferred_element_type=jnp.float32)
        o += jnp.dot(ht_ref[jc], w2r_ref[pl.ds(jc * 128, 128), :],
                     preferred_element_type=jnp.float32)
    out_ref[...] = o


_mm1 = pl.pallas_call(
    _mm1_body,
    grid=(GRID,),
    in_specs=[
        pl.BlockSpec((DIN // 128, NB, 128), lambda i: (0, i, 0)),
        pl.BlockSpec((1, NB, 128), lambda i: (2, i, 0)),
        pl.BlockSpec((NB, DIN), lambda i: (i, 0)),
        pl.BlockSpec((DIN, H), lambda i: (0, 0)),
        pl.BlockSpec((1, H), lambda i: (0, 0)),
        pl.BlockSpec((DIN, H), lambda i: (0, 0)),
    ],
    out_specs=pl.BlockSpec((H // 128, NB, 128), lambda i: (0, i, 0)),
    out_shape=jax.ShapeDtypeStruct((H // 128, N, 128), jnp.float32),
)

_mm2 = pl.pallas_call(
    _mm2_body,
    grid=(GRID,),
    in_specs=[
        pl.BlockSpec((H // 128, NB, 128), lambda i: (0, i, 0)),
        pl.BlockSpec((1, NB, 128), lambda i: (2, i, 0)),
        pl.BlockSpec((H // 128, NB, 128), lambda i: (0, i, 0)),
        pl.BlockSpec((H, H), lambda i: (0, 0)),
        pl.BlockSpec((1, H), lambda i: (0, 0)),
        pl.BlockSpec((H, H), lambda i: (0, 0)),
    ],
    out_specs=pl.BlockSpec((NB, H), lambda i: (i, 0)),
    out_shape=jax.ShapeDtypeStruct((N, H), jnp.float32),
)


def kernel(x, edge_index, W1l, b1, W1r, W2l, b2, W2r):
    src = edge_index[0].astype(jnp.int32)
    dst = edge_index[1].astype(jnp.int32)
    pad = E_PAD - E
    src_pad = jnp.concatenate([src, jnp.zeros((pad,), jnp.int32)])
    dst_pad = jnp.concatenate([dst, jnp.full((pad,), N, jnp.int32)])
    src_all1 = (src_pad[None, :]
                + (jnp.arange(DIN // 128, dtype=jnp.int32) * N)[:, None]).reshape(-1)
    src_all2 = (src_pad[None, :]
                + (jnp.arange(H // 128, dtype=jnp.int32) * N)[:, None]).reshape(-1)

    x_t = x.reshape(N, DIN // 128, 128).transpose(1, 0, 2).reshape(-1, 128)
    zrows = jnp.zeros((ROWS_PER_TILE, 128), jnp.float32)
    ones = jnp.ones((CH, 128), jnp.float32)

    agg1 = _make_agg(True)(x_t, src_all1, dst_pad, zrows, ones)
    h_t = _mm1(agg1, agg1, x, W1l, b1.reshape(1, H), W1r)
    agg2 = _make_agg(False)(h_t.reshape(-1, 128), src_all2, dst_pad, zrows, ones)
    out = _mm2(agg2, agg1, h_t, W2l, b2.reshape(1, H), W2r)
    return out


# speedup vs baseline: 3.2020x; 1.4296x over previous
import functools

import jax
import jax.numpy as jnp
from jax import lax
from jax.experimental import pallas as pl
from jax.experimental.pallas import tpu as pltpu
from jax.experimental.pallas import tpu_sc as plsc

N = 10000
E = 160000
DIN = 256
H = 512

NC = 2
NS = 16
CH = 128
PER_TILE = 10240
E_PAD = PER_TILE * NS
CHUNKS = PER_TILE // CH
ROWS_PER_TILE = 632
N_ACC = ROWS_PER_TILE * NS
NROW = 2
NSRC = 4


def _make_agg(with_counts):
    mesh = plsc.VectorSubcoreMesh(
        core_axis_name="c", subcore_axis_name="s", num_cores=NC, num_subcores=NS
    )
    out_type = jax.ShapeDtypeStruct((4, N_ACC, 128), jnp.float32)
    scratch = (
        [pltpu.VMEM_SHARED((N_ACC, 128), jnp.float32)]
        + [pltpu.VMEM((CHUNKS, CH), jnp.int32)]
        + [pltpu.VMEM((CH,), jnp.int32)] * NSRC
        + [pltpu.VMEM((CH, 128), jnp.float32)] * NROW
        + [pltpu.SemaphoreType.DMA] * (NSRC + NROW)
    )

    def body(table, src_all, dst, zrows, ones, agg_out, acc,
             dst_vm, *ring):
        srcs = ring[:NSRC]
        rows = ring[NSRC:NSRC + NROW]
        isem = ring[NSRC + NROW:2 * NSRC + NROW]
        gsem = ring[2 * NSRC + NROW:]
        c = lax.axis_index("c")
        s = lax.axis_index("s")
        row0 = s * ROWS_PER_TILE

        pltpu.sync_copy(dst.at[s], dst_vm)

        def fire_idx(base, k, slot):
            pltpu.async_copy(
                src_all.at[pl.ds(base + k * CH, CH)], srcs[slot], isem[slot]
            )

        def fire_gather(slot_i, slot_r):
            pltpu.async_copy(table.at[srcs[slot_i]], rows[slot_r],
                             gsem[slot_r])

        def wait_idx(slot):
            pltpu.make_async_copy(src_all.at[pl.ds(0, CH)], srcs[slot],
                                  isem[slot]).wait()

        def wait_gather(slot_i, slot_r):
            pltpu.make_async_copy(table.at[srcs[slot_i]], rows[slot_r],
                                  gsem[slot_r]).wait()

        for p in range(2):
            count_pass = with_counts and p == 1
            if with_counts:
                j_table = c
                j_out = c + 2 * p
            else:
                j_table = c * 2 + p
                j_out = j_table
            pltpu.sync_copy(zrows, acc.at[pl.ds(row0, ROWS_PER_TILE)])
            if count_pass:
                pltpu.sync_copy(ones, rows[0])
                plsc.subcore_barrier()

                def cchunk(k, carry):
                    pltpu.sync_copy(rows[0], acc.at[dst_vm.at[k]], add=True)
                    return carry

                lax.fori_loop(c * (CHUNKS // 2), (c + 1) * (CHUNKS // 2),
                              cchunk, 0)
            else:
                base = j_table * E_PAD + s * PER_TILE
                plsc.subcore_barrier()

                fire_idx(base, 0, 0)
                fire_idx(base, 1, 1)
                wait_idx(0)
                fire_gather(0, 0)

                def group(k0, carry):
                    for b in range(NSRC):
                        k = k0 * NSRC + b
                        kw = lax.rem(k + 2, CHUNKS)
                        fire_idx(base, kw, (b + 2) % NSRC)
                        wait_idx((b + 1) % NSRC)
                        fire_gather((b + 1) % NSRC, (b + 1) % NROW)
                        wait_gather(b % NSRC, b % NROW)
                        pltpu.sync_copy(rows[b % NROW], acc.at[dst_vm.at[k]],
                                        add=True)
                    return carry

                lax.fori_loop(0, CHUNKS // NSRC, group, 0)
                wait_idx((CHUNKS + 1) % NSRC)
                wait_gather(CHUNKS % NSRC, CHUNKS % NROW)
            plsc.subcore_barrier()

            pltpu.sync_copy(
                acc.at[pl.ds(row0, ROWS_PER_TILE)],
                agg_out.at[j_out, pl.ds(row0, ROWS_PER_TILE)],
            )
            if p == 0:
                plsc.subcore_barrier()

    return pl.kernel(
        body,
        out_type=out_type,
        mesh=mesh,
        scratch_types=tuple(scratch),
    )


_make_agg = functools.lru_cache(maxsize=None)(_make_agg)


NB = 400
GRID = N // NB


def _mm1_body(agg_ref, cnt_ref, x_ref, w1l_ref, b1_ref, w1r_ref, ht_ref):
    cnt = cnt_ref[0, :, 0:1] + cnt_ref[1, :, 0:1]
    inv = 1.0 / jnp.maximum(cnt, 1.0)
    h = jnp.dot(x_ref[...], w1r_ref[...], preferred_element_type=jnp.float32)
    h += b1_ref[...]
    for jc in range(DIN // 128):
        a = agg_ref[jc] * inv
        h += jnp.dot(a, w1l_ref[pl.ds(jc * 128, 128), :],
                     preferred_element_type=jnp.float32)
    h = jnp.maximum(h, 0.0)
    for jc in range(H // 128):
        ht_ref[jc] = h[:, jc * 128:(jc + 1) * 128]


def _mm2_body(agg_ref, cnt_ref, ht_ref, w2l_ref, b2_ref, w2r_ref, out_ref):
    cnt = cnt_ref[0, :, 0:1] + cnt_ref[1, :, 0:1]
    inv = 1.0 / jnp.maximum(cnt, 1.0)
    o = jnp.broadcast_to(b2_ref[...], (NB, H)).astype(jnp.float32)
    for jc in range(H // 128):
        a = agg_ref[jc] * inv
        o += jnp.dot(a, w2l_ref[pl.ds(jc * 128, 128), :],
                     preferred_element_type=jnp.float32)
        o += jnp.dot(ht_ref[jc], w2r_ref[pl.ds(jc * 128, 128), :],
                     preferred_element_type=jnp.float32)
    out_ref[...] = o


_mm1 = pl.pallas_call(
    _mm1_body,
    grid=(GRID,),
    in_specs=[
        pl.BlockSpec((DIN // 128, NB, 128), lambda i: (0, i, 0)),
        pl.BlockSpec((2, NB, 128), lambda i: (1, i, 0)),
        pl.BlockSpec((NB, DIN), lambda i: (i, 0)),
        pl.BlockSpec((DIN, H), lambda i: (0, 0)),
        pl.BlockSpec((1, H), lambda i: (0, 0)),
        pl.BlockSpec((DIN, H), lambda i: (0, 0)),
    ],
    out_specs=pl.BlockSpec((H // 128, NB, 128), lambda i: (0, i, 0)),
    out_shape=jax.ShapeDtypeStruct((H // 128, N, 128), jnp.float32),
)

_mm2 = pl.pallas_call(
    _mm2_body,
    grid=(GRID,),
    in_specs=[
        pl.BlockSpec((H // 128, NB, 128), lambda i: (0, i, 0)),
        pl.BlockSpec((2, NB, 128), lambda i: (1, i, 0)),
        pl.BlockSpec((H // 128, NB, 128), lambda i: (0, i, 0)),
        pl.BlockSpec((H, H), lambda i: (0, 0)),
        pl.BlockSpec((1, H), lambda i: (0, 0)),
        pl.BlockSpec((H, H), lambda i: (0, 0)),
    ],
    out_specs=pl.BlockSpec((NB, H), lambda i: (i, 0)),
    out_shape=jax.ShapeDtypeStruct((N, H), jnp.float32),
)


def kernel(x, edge_index, W1l, b1, W1r, W2l, b2, W2r):
    src = edge_index[0].astype(jnp.int32)
    dst = edge_index[1].astype(jnp.int32)
    pad = E_PAD - E
    src_pad = jnp.concatenate([src, jnp.zeros((pad,), jnp.int32)])
    dst_pad = jnp.concatenate([dst, jnp.full((pad,), N, jnp.int32)])
    src_all1 = (src_pad[None, :]
                + (jnp.arange(DIN // 128, dtype=jnp.int32) * N)[:, None]).reshape(-1)
    src_all2 = (src_pad[None, :]
                + (jnp.arange(H // 128, dtype=jnp.int32) * N)[:, None]).reshape(-1)
    dst_3d = dst_pad.reshape(NS, CHUNKS, CH)

    x_t = x.reshape(N, DIN // 128, 128).transpose(1, 0, 2).reshape(-1, 128)
    zrows = jnp.zeros((ROWS_PER_TILE, 128), jnp.float32)
    ones = jnp.ones((CH, 128), jnp.float32)

    agg1 = _make_agg(True)(x_t, src_all1, dst_3d, zrows, ones)
    h_t = _mm1(agg1, agg1, x, W1l, b1.reshape(1, H), W1r)
    agg2 = _make_agg(False)(h_t.reshape(-1, 128), src_all2, dst_3d, zrows, ones)
    out = _mm2(agg2, agg1, h_t, W2l, b2.reshape(1, H), W2r)
    return out


# speedup vs baseline: 3.2284x; 1.0082x over previous
import functools

import jax
import jax.numpy as jnp
from jax import lax
from jax.experimental import pallas as pl
from jax.experimental.pallas import tpu as pltpu
from jax.experimental.pallas import tpu_sc as plsc

N = 10000
E = 160000
DIN = 256
H = 512

NC = 2
NS = 16
CH = 128
PER_TILE = 10240
E_PAD = PER_TILE * NS
CHUNKS = PER_TILE // CH
ROWS_PER_TILE = 632
N_ACC = ROWS_PER_TILE * NS
NROW = 2
NSRC = 4


def _make_agg(with_counts):
    mesh = plsc.VectorSubcoreMesh(
        core_axis_name="c", subcore_axis_name="s", num_cores=NC, num_subcores=NS
    )
    out_type = jax.ShapeDtypeStruct((4, N_ACC, 128), jnp.float32)
    scratch = (
        [pltpu.VMEM_SHARED((N_ACC, 128), jnp.float32)]
        + [pltpu.VMEM((CHUNKS, CH), jnp.int32)]
        + [pltpu.VMEM((CH,), jnp.int32)] * NSRC
        + [pltpu.VMEM((CH, 128), jnp.float32)] * NROW
        + [pltpu.SemaphoreType.DMA] * (NSRC + 2 * NROW)
    )

    def body(table, src_all, dst, zrows, ones, agg_out, acc,
             dst_vm, *ring):
        srcs = ring[:NSRC]
        rows = ring[NSRC:NSRC + NROW]
        isem = ring[NSRC + NROW:2 * NSRC + NROW]
        gsem = ring[2 * NSRC + NROW:2 * NSRC + 2 * NROW]
        ssem = ring[2 * NSRC + 2 * NROW:]
        c = lax.axis_index("c")
        s = lax.axis_index("s")
        row0 = s * ROWS_PER_TILE

        pltpu.sync_copy(dst.at[s], dst_vm)

        def fire_idx(base, k, slot):
            pltpu.async_copy(
                src_all.at[pl.ds(base + k * CH, CH)], srcs[slot], isem[slot]
            )

        def fire_gather(slot_i, slot_r):
            pltpu.async_copy(table.at[srcs[slot_i]], rows[slot_r],
                             gsem[slot_r])

        def wait_idx(slot):
            pltpu.make_async_copy(src_all.at[pl.ds(0, CH)], srcs[slot],
                                  isem[slot]).wait()

        def wait_gather(slot_i, slot_r):
            pltpu.make_async_copy(table.at[srcs[slot_i]], rows[slot_r],
                                  gsem[slot_r]).wait()

        def fire_scat(k, slot_r):
            pltpu.async_copy(rows[slot_r], acc.at[dst_vm.at[k]],
                             ssem[slot_r], add=True)

        def wait_scat(slot_r):
            pltpu.make_async_copy(rows[slot_r], acc.at[dst_vm.at[0]],
                                  ssem[slot_r]).wait()

        for p in range(2):
            count_pass = with_counts and p == 1
            if with_counts:
                j_table = c
                j_out = c + 2 * p
            else:
                j_table = c * 2 + p
                j_out = j_table
            pltpu.sync_copy(zrows, acc.at[pl.ds(row0, ROWS_PER_TILE)])
            if count_pass:
                pltpu.sync_copy(ones, rows[0])
                plsc.subcore_barrier()

                def cfire(k, carry):
                    fire_scat(k, 0)
                    return carry

                lax.fori_loop(c * (CHUNKS // 2), (c + 1) * (CHUNKS // 2),
                              cfire, 0)

                def cdrain(_, carry):
                    wait_scat(0)
                    return carry

                lax.fori_loop(0, CHUNKS // 2, cdrain, 0)
            else:
                base = j_table * E_PAD + s * PER_TILE
                plsc.subcore_barrier()

                def step(k, b, first=False, wrap=False):
                    kw = (k + 2) % CHUNKS if wrap else k + 2
                    fire_idx(base, kw, (b + 2) % NSRC)
                    if not first:
                        wait_scat((b + 1) % NROW)
                    wait_idx((b + 1) % NSRC)
                    fire_gather((b + 1) % NSRC, (b + 1) % NROW)
                    wait_gather(b % NSRC, b % NROW)
                    fire_scat(k, b % NROW)

                fire_idx(base, 0, 0)
                fire_idx(base, 1, 1)
                wait_idx(0)
                fire_gather(0, 0)
                step(0, 0, first=True)

                def group(k0, carry):
                    for b in range(NSRC):
                        step(k0 * NSRC + 1 + b, 1 + b)
                    return carry

                lax.fori_loop(0, (CHUNKS - 4) // NSRC, group, 0)
                for k in (CHUNKS - 3, CHUNKS - 2, CHUNKS - 1):
                    step(k, k, wrap=True)
                wait_scat((CHUNKS - 1) % NROW)
                wait_idx((CHUNKS + 1) % NSRC)
                wait_gather(CHUNKS % NSRC, CHUNKS % NROW)
            plsc.subcore_barrier()

            pltpu.sync_copy(
                acc.at[pl.ds(row0, ROWS_PER_TILE)],
                agg_out.at[j_out, pl.ds(row0, ROWS_PER_TILE)],
            )
            if p == 0:
                plsc.subcore_barrier()

    return pl.kernel(
        body,
        out_type=out_type,
        mesh=mesh,
        scratch_types=tuple(scratch),
    )


_make_agg = functools.lru_cache(maxsize=None)(_make_agg)


NB = 400
GRID = N // NB


def _mm1_body(agg_ref, cnt_ref, x_ref, w1l_ref, b1_ref, w1r_ref, ht_ref):
    cnt = cnt_ref[0, :, 0:1] + cnt_ref[1, :, 0:1]
    inv = 1.0 / jnp.maximum(cnt, 1.0)
    h = jnp.dot(x_ref[...], w1r_ref[...], preferred_element_type=jnp.float32)
    h += b1_ref[...]
    for jc in range(DIN // 128):
        a = agg_ref[jc] * inv
        h += jnp.dot(a, w1l_ref[pl.ds(jc * 128, 128), :],
                     preferred_element_type=jnp.float32)
    h = jnp.maximum(h, 0.0)
    for jc in range(H // 128):
        ht_ref[jc] = h[:, jc * 128:(jc + 1) * 128]


def _mm2_body(agg_ref, cnt_ref, ht_ref, w2l_ref, b2_ref, w2r_ref, out_ref):
    cnt = cnt_ref[0, :, 0:1] + cnt_ref[1, :, 0:1]
    inv = 1.0 / jnp.maximum(cnt, 1.0)
    o = jnp.broadcast_to(b2_ref[...], (NB, H)).astype(jnp.float32)
    for jc in range(H // 128):
        a = agg_ref[jc] * inv
        o += jnp.dot(a, w2l_ref[pl.ds(jc * 128, 128), :],
                     preferred_element_type=jnp.float32)
        o += jnp.dot(ht_ref[jc], w2r_ref[pl.ds(jc * 128, 128), :],
                     preferred_element_type=jnp.float32)
    out_ref[...] = o


_mm1 = pl.pallas_call(
    _mm1_body,
    grid=(GRID,),
    in_specs=[
        pl.BlockSpec((DIN // 128, NB, 128), lambda i: (0, i, 0)),
        pl.BlockSpec((2, NB, 128), lambda i: (1, i, 0)),
        pl.BlockSpec((NB, DIN), lambda i: (i, 0)),
        pl.BlockSpec((DIN, H), lambda i: (0, 0)),
        pl.BlockSpec((1, H), lambda i: (0, 0)),
        pl.BlockSpec((DIN, H), lambda i: (0, 0)),
    ],
    out_specs=pl.BlockSpec((H // 128, NB, 128), lambda i: (0, i, 0)),
    out_shape=jax.ShapeDtypeStruct((H // 128, N, 128), jnp.float32),
)

_mm2 = pl.pallas_call(
    _mm2_body,
    grid=(GRID,),
    in_specs=[
        pl.BlockSpec((H // 128, NB, 128), lambda i: (0, i, 0)),
        pl.BlockSpec((2, NB, 128), lambda i: (1, i, 0)),
        pl.BlockSpec((H // 128, NB, 128), lambda i: (0, i, 0)),
        pl.BlockSpec((H, H), lambda i: (0, 0)),
        pl.BlockSpec((1, H), lambda i: (0, 0)),
        pl.BlockSpec((H, H), lambda i: (0, 0)),
    ],
    out_specs=pl.BlockSpec((NB, H), lambda i: (i, 0)),
    out_shape=jax.ShapeDtypeStruct((N, H), jnp.float32),
)


def kernel(x, edge_index, W1l, b1, W1r, W2l, b2, W2r):
    src = edge_index[0].astype(jnp.int32)
    dst = edge_index[1].astype(jnp.int32)
    pad = E_PAD - E
    src_pad = jnp.concatenate([src, jnp.zeros((pad,), jnp.int32)])
    dst_pad = jnp.concatenate([dst, jnp.full((pad,), N, jnp.int32)])
    src_all1 = (src_pad[None, :]
                + (jnp.arange(DIN // 128, dtype=jnp.int32) * N)[:, None]).reshape(-1)
    src_all2 = (src_pad[None, :]
                + (jnp.arange(H // 128, dtype=jnp.int32) * N)[:, None]).reshape(-1)
    dst_3d = dst_pad.reshape(NS, CHUNKS, CH)

    x_t = x.reshape(N, DIN // 128, 128).transpose(1, 0, 2).reshape(-1, 128)
    zrows = jnp.zeros((ROWS_PER_TILE, 128), jnp.float32)
    ones = jnp.ones((CH, 128), jnp.float32)

    agg1 = _make_agg(True)(x_t, src_all1, dst_3d, zrows, ones)
    h_t = _mm1(agg1, agg1, x, W1l, b1.reshape(1, H), W1r)
    agg2 = _make_agg(False)(h_t.reshape(-1, 128), src_all2, dst_3d, zrows, ones)
    out = _mm2(agg2, agg1, h_t, W2l, b2.reshape(1, H), W2r)
    return out
